# baseline (speedup 1.0000x reference)
import functools

import jax
import jax.numpy as jnp
from jax import lax
from jax.experimental import pallas as pl
from jax.experimental.pallas import tpu as pltpu
from jax.experimental.pallas import tpu_sc as plsc

B = 16384
L = 50
NUM_WORKERS = 32
ROWS_PER_W = B // NUM_WORKERS
WORDS_PER_W = ROWS_PER_W * L
LANES = 16
VREGS_PER_W = WORDS_PER_W // LANES

_MESH = plsc.VectorSubcoreMesh(core_axis_name="c", subcore_axis_name="s")


@functools.partial(
    pl.kernel,
    out_type=jax.ShapeDtypeStruct((B * L,), jnp.float32),
    mesh=_MESH,
    scratch_types=[
        pltpu.VMEM((WORDS_PER_W + LANES,), jnp.int32),
        pltpu.VMEM((ROWS_PER_W,), jnp.float32),
        pltpu.VMEM((LANES,), jnp.float32),
        pltpu.VMEM((WORDS_PER_W,), jnp.float32),
        pltpu.SemaphoreType.DMA,
        pltpu.SemaphoreType.DMA,
        pltpu.SemaphoreType.DMA,
    ],
    compiler_params=pltpu.CompilerParams(needs_layout_passes=False),
)
def _word_reward_sc(trie_hbm, ifw_hbm, tab_hbm, out_hbm,
                    trie_v, ifw_v, tab_v, out_v, sem_in, sem_out, sem_misc):
    wid = lax.axis_index("s") * 2 + lax.axis_index("c")
    base = wid * WORDS_PER_W

    NCH = 2
    CWORDS = WORDS_PER_W // NCH
    CVREGS = CWORDS // LANES

    in_copies = [pltpu.async_copy(
        trie_hbm.at[pl.ds(base + c * CWORDS, CWORDS)],
        trie_v.at[pl.ds(c * CWORDS, CWORDS)], sem_in) for c in range(1)]
    c2 = pltpu.async_copy(ifw_hbm.at[pl.ds(wid * ROWS_PER_W, ROWS_PER_W)],
                          ifw_v, sem_misc)
    c3 = pltpu.async_copy(tab_hbm, tab_v.at[pl.ds(0, 5)], sem_misc)
    in_copies.append(pltpu.async_copy(
        trie_hbm.at[pl.ds(base + 1 * CWORDS, CWORDS)],
        trie_v.at[pl.ds(1 * CWORDS, CWORDS)], sem_in))
    trie_v[pl.ds(WORDS_PER_W, LANES)] = jnp.full((LANES,), -1, jnp.int32)
    c2.wait()
    c3.wait()

    lane = lax.iota(jnp.int32, LANES)
    out_copies = []
    for c in range(NCH):
        in_copies[c].wait()
        if c + 2 < NCH:
            in_copies.append(pltpu.async_copy(
                trie_hbm.at[pl.ds(base + (c + 2) * CWORDS, CWORDS)],
                trie_v.at[pl.ds((c + 2) * CWORDS, CWORDS)], sem_in))

        @plsc.parallel_loop(c * CVREGS, (c + 1) * CVREGS, unroll=8)
        def step(i):
            off = i * LANES
            t = trie_v[pl.ds(off, LANES)]
            tn = trie_v[pl.ds(off + 1, LANES)]
            w = plsc.load_gather(tab_v, [t + 1])
            fidx = lane + off
            row = (fidx * 5243) >> 18
            bonus = plsc.load_gather(ifw_v, [row])
            islast = jnp.logical_and(t != -1, tn == -1)
            out_v[pl.ds(off, LANES)] = w + bonus * islast.astype(jnp.float32)

        out_copies.append(pltpu.async_copy(
            out_v.at[pl.ds(c * CWORDS, CWORDS)],
            out_hbm.at[pl.ds(base + c * CWORDS, CWORDS)], sem_out))
    for oc in out_copies:
        oc.wait()


def kernel(token_words, trie_values, is_full_word, reward_mapping_values):
    del token_words
    out_flat = _word_reward_sc(trie_values.reshape(B * L), is_full_word,
                               reward_mapping_values)
    return out_flat.reshape(B, L)


# speedup vs baseline: 1.0057x; 1.0057x over previous
import functools

import jax
import jax.numpy as jnp
from jax import lax
from jax.experimental import pallas as pl
from jax.experimental.pallas import tpu as pltpu
from jax.experimental.pallas import tpu_sc as plsc

B = 16384
L = 50
NUM_WORKERS = 32
ROWS_PER_W = B // NUM_WORKERS
WORDS_PER_W = ROWS_PER_W * L
LANES = 16
VREGS_PER_W = WORDS_PER_W // LANES

_MESH = plsc.VectorSubcoreMesh(core_axis_name="c", subcore_axis_name="s")


@functools.partial(
    pl.kernel,
    out_type=jax.ShapeDtypeStruct((B * L,), jnp.float32),
    mesh=_MESH,
    scratch_types=[
        pltpu.VMEM((WORDS_PER_W + LANES,), jnp.int32),
        pltpu.VMEM((ROWS_PER_W,), jnp.float32),
        pltpu.VMEM((LANES,), jnp.float32),
        pltpu.VMEM((WORDS_PER_W,), jnp.float32),
        pltpu.SemaphoreType.DMA,
        pltpu.SemaphoreType.DMA,
        pltpu.SemaphoreType.DMA,
    ],
    compiler_params=pltpu.CompilerParams(needs_layout_passes=False),
)
def _word_reward_sc(trie_hbm, ifw_hbm, tab_hbm, out_hbm,
                    trie_v, ifw_v, tab_v, out_v, sem_in, sem_out, sem_misc):
    wid = lax.axis_index("s") * 2 + lax.axis_index("c")
    base = wid * WORDS_PER_W

    NCH = 4
    CWORDS = WORDS_PER_W // NCH
    CVREGS = CWORDS // LANES

    in_copies = [pltpu.async_copy(
        trie_hbm.at[pl.ds(base + c * CWORDS, CWORDS)],
        trie_v.at[pl.ds(c * CWORDS, CWORDS)], sem_in) for c in range(1)]
    c2 = pltpu.async_copy(ifw_hbm.at[pl.ds(wid * ROWS_PER_W, ROWS_PER_W)],
                          ifw_v, sem_misc)
    c3 = pltpu.async_copy(tab_hbm, tab_v.at[pl.ds(0, 5)], sem_misc)
    in_copies.append(pltpu.async_copy(
        trie_hbm.at[pl.ds(base + 1 * CWORDS, CWORDS)],
        trie_v.at[pl.ds(1 * CWORDS, CWORDS)], sem_in))
    trie_v[pl.ds(WORDS_PER_W, LANES)] = jnp.full((LANES,), -1, jnp.int32)
    c2.wait()
    c3.wait()

    lane = lax.iota(jnp.int32, LANES)
    out_copies = []
    for c in range(NCH):
        in_copies[c].wait()
        if c + 2 < NCH:
            in_copies.append(pltpu.async_copy(
                trie_hbm.at[pl.ds(base + (c + 2) * CWORDS, CWORDS)],
                trie_v.at[pl.ds((c + 2) * CWORDS, CWORDS)], sem_in))

        @plsc.parallel_loop(c * CVREGS, (c + 1) * CVREGS, unroll=8)
        def step(i):
            off = i * LANES
            t = trie_v[pl.ds(off, LANES)]
            tn = trie_v[pl.ds(off + 1, LANES)]
            w = plsc.load_gather(tab_v, [t + 1])
            fidx = lane + off
            row = (fidx * 5243) >> 18
            bonus = plsc.load_gather(ifw_v, [row])
            islast = jnp.logical_and(t != -1, tn == -1)
            out_v[pl.ds(off, LANES)] = w + bonus * islast.astype(jnp.float32)

        out_copies.append(pltpu.async_copy(
            out_v.at[pl.ds(c * CWORDS, CWORDS)],
            out_hbm.at[pl.ds(base + c * CWORDS, CWORDS)], sem_out))
    for oc in out_copies:
        oc.wait()


def kernel(token_words, trie_values, is_full_word, reward_mapping_values):
    del token_words
    out_flat = _word_reward_sc(trie_values.reshape(B * L), is_full_word,
                               reward_mapping_values)
    return out_flat.reshape(B, L)
